# Initial kernel scaffold; baseline (speedup 1.0000x reference)
import jax
import jax.numpy as jnp
from jax.experimental import pallas as pl


def kernel(pro_x, pro_edge_index, pro_edge_weight, pro_batch, global_rna, local_rna, emb1, emb2, conv1_w, conv1_b, conv2_w, conv2_b, fc_xr_w, fc_xr_b, gcn_w1, gcn_b1, gcn_w2, gcn_b2, gcn_w3, gcn_b3, bn1_w, bn1_b, bn2_w, bn2_b, bn3_w, bn3_b, fc_g1_w, fc_g1_b, fc_g2_w, fc_g2_b):
    raise NotImplementedError("write your pallas kernel here")


# speedup vs baseline: 3.1791x; 3.1791x over previous
import functools
import jax
import jax.numpy as jnp
from jax.experimental import pallas as pl
from jax.experimental.pallas import tpu as pltpu

_N = 50000
_E = 800000
_B = 32
_MAX_LEN = 3000
_LOCAL_LEN = 2998
_EMBED = 128
_NF = 32
_OUT = 128
_BM = 2000


def _onehot_body(g1_ref, g2_ref, w1_ref, w2_ref, c1_ref, c2_ref):
  g1 = g1_ref[0]
  oh1 = (jax.lax.broadcasted_iota(jnp.int32, (8, _MAX_LEN), 0)
         == g1).astype(jnp.float32)
  c1_ref[0] = jnp.dot(oh1, w1_ref[...], preferred_element_type=jnp.float32)
  g2 = g2_ref[0]
  oh2 = (jax.lax.broadcasted_iota(jnp.int32, (72, _LOCAL_LEN), 0)
         == g2).astype(jnp.float32)
  c2_ref[0] = jnp.dot(oh2, w2_ref[...], preferred_element_type=jnp.float32)


def _rna_counts(g1, g2, w1p, w2p):
  return pl.pallas_call(
      _onehot_body,
      grid=(_B,),
      in_specs=[
          pl.BlockSpec((1, 1, _MAX_LEN), lambda i: (i, 0, 0)),
          pl.BlockSpec((1, 1, _LOCAL_LEN), lambda i: (i, 0, 0)),
          pl.BlockSpec((_MAX_LEN, 256), lambda i: (0, 0)),
          pl.BlockSpec((_LOCAL_LEN, 256), lambda i: (0, 0)),
      ],
      out_specs=[
          pl.BlockSpec((1, 8, 256), lambda i: (i, 0, 0)),
          pl.BlockSpec((1, 72, 256), lambda i: (i, 0, 0)),
      ],
      out_shape=[
          jax.ShapeDtypeStruct((_B, 8, 256), jnp.float32),
          jax.ShapeDtypeStruct((_B, 72, 256), jnp.float32),
      ],
  )(g1.reshape(_B, 1, _MAX_LEN), g2.reshape(_B, 1, _LOCAL_LEN), w1p, w2p)


def _mix_body(c1_ref, c2_ref, e1_ref, e2_ref, fw_ref, b_ref, cb_ref, o_ref):
  m1 = jnp.dot(e1_ref[...], fw_ref[...],
               preferred_element_type=jnp.float32)
  m1 = m1.reshape(64 * 32, 128)
  xrg = jnp.dot(c1_ref[...], m1, preferred_element_type=jnp.float32)
  m2 = jnp.dot(e2_ref[...], fw_ref[...],
               preferred_element_type=jnp.float32)
  m2 = m2.reshape(576 * 32, 128)
  xrl = jnp.dot(c2_ref[...], m2, preferred_element_type=jnp.float32)
  fs = jnp.sum(fw_ref[...], axis=0).reshape(_NF, _OUT)
  cbt = jnp.dot(cb_ref[...], fs, preferred_element_type=jnp.float32)
  o_ref[...] = ((xrg + xrl) * 0.5 + b_ref[...]
                + 0.5 * (cbt[0:1, :] + cbt[1:2, :]))


def _rna_mix(c1f, c2f, e1p, e2p, fw_oh, fc_xr_b, cb):
  return pl.pallas_call(
      _mix_body,
      out_shape=jax.ShapeDtypeStruct((_B, _OUT), jnp.float32),
  )(c1f, c2f, e1p, e2p, fw_oh, fc_xr_b.reshape(1, _OUT), cb)


def _xs_body(x_ref, w_ref, dinv_ref, xs_ref):
  xw = jnp.dot(x_ref[...], w_ref[...], preferred_element_type=jnp.float32)
  xs_ref[...] = xw * dinv_ref[...]


def _stage_xs(x, wT, dinv):
  fin, fout = wT.shape
  return pl.pallas_call(
      _xs_body,
      grid=(_N // _BM,),
      in_specs=[
          pl.BlockSpec((_BM, fin), lambda i: (i, 0)),
          pl.BlockSpec((fin, fout), lambda i: (0, 0)),
          pl.BlockSpec((_BM, 1), lambda i: (i, 0)),
      ],
      out_specs=pl.BlockSpec((_BM, fout), lambda i: (i, 0)),
      out_shape=jax.ShapeDtypeStruct((_N, fout), jnp.float32),
  )(x, wT, dinv)


def _post_body(agg_ref, xs_ref, dinv_ref, b_ref, bnw_ref, bnb_ref, o_ref):
  y = (agg_ref[...] + xs_ref[...]) * dinv_ref[...] + b_ref[...]
  y = y * (1.0 / jnp.sqrt(1.0 + 1e-05)) * bnw_ref[...] + bnb_ref[...]
  o_ref[...] = jnp.maximum(y, 0.0)


def _stage_post(agg, xs, dinv, b, bnw, bnb):
  f = agg.shape[1]
  return pl.pallas_call(
      _post_body,
      grid=(_N // _BM,),
      in_specs=[
          pl.BlockSpec((_BM, f), lambda i: (i, 0)),
          pl.BlockSpec((_BM, f), lambda i: (i, 0)),
          pl.BlockSpec((_BM, 1), lambda i: (i, 0)),
          pl.BlockSpec((1, f), lambda i: (0, 0)),
          pl.BlockSpec((1, f), lambda i: (0, 0)),
          pl.BlockSpec((1, f), lambda i: (0, 0)),
      ],
      out_specs=pl.BlockSpec((_BM, f), lambda i: (i, 0)),
      out_shape=jax.ShapeDtypeStruct((_N, f), jnp.float32),
  )(agg, xs, dinv, b.reshape(1, f), bnw.reshape(1, f), bnb.reshape(1, f))


def _pool_body(batch_ref, x_ref, acc_ref):
  i = pl.program_id(0)

  @pl.when(i == 0)
  def _():
    acc_ref[...] = jnp.zeros_like(acc_ref)

  bid = batch_ref[0]
  oh = (jax.lax.broadcasted_iota(jnp.int32, (_B, _BM), 0)
        == bid).astype(jnp.float32)
  xcat = jnp.concatenate(
      [x_ref[...], jnp.ones((_BM, 8), jnp.float32)], axis=1)
  acc_ref[...] += jnp.dot(oh, xcat, preferred_element_type=jnp.float32)


def _pool(batch, x3):
  f = x3.shape[1]
  return pl.pallas_call(
      _pool_body,
      grid=(_N // _BM,),
      in_specs=[
          pl.BlockSpec((1, 1, _BM), lambda i: (i, 0, 0)),
          pl.BlockSpec((_BM, f), lambda i: (i, 0)),
      ],
      out_specs=pl.BlockSpec((_B, f + 8), lambda i: (0, 0)),
      out_shape=jax.ShapeDtypeStruct((_B, f + 8), jnp.float32),
  )(batch.reshape(_N // _BM, 1, _BM), x3)


def _head_body(s_ref, w1_ref, b1_ref, w2_ref, b2_ref, o_ref):
  f = 132
  sums = s_ref[:, :f]
  cnt = jnp.maximum(s_ref[:, f:f + 1], 1.0)
  mean = sums / cnt
  h = jnp.dot(mean, w1_ref[...], preferred_element_type=jnp.float32)
  h = jnp.maximum(h + b1_ref[...], 0.0)
  o_ref[...] = jnp.dot(h, w2_ref[...],
                       preferred_element_type=jnp.float32) + b2_ref[...]


def _head(sums_cat, w1T, b1, w2T, b2):
  return pl.pallas_call(
      _head_body,
      out_shape=jax.ShapeDtypeStruct((_B, _OUT), jnp.float32),
  )(sums_cat, w1T, b1.reshape(1, -1), w2T, b2.reshape(1, -1))


def kernel(pro_x, pro_edge_index, pro_edge_weight, pro_batch, global_rna,
           local_rna, emb1, emb2, conv1_w, conv1_b, conv2_w, conv2_b,
           fc_xr_w, fc_xr_b, gcn_w1, gcn_b1, gcn_w2, gcn_b2, gcn_w3, gcn_b3,
           bn1_w, bn1_b, bn2_w, bn2_b, bn3_w, bn3_b, fc_g1_w, fc_g1_b,
           fc_g2_w, fc_g2_b):
  w1p = conv1_w.transpose(1, 2, 0).reshape(_MAX_LEN, 8 * _NF)
  w2p = conv2_w.transpose(1, 2, 0).reshape(_LOCAL_LEN, 8 * _NF)
  c1, c2 = _rna_counts(global_rna, local_rna, w1p, w2p)
  emb1p = jnp.zeros((8, _EMBED), jnp.float32).at[:5].set(emb1)
  emb2p = jnp.zeros((72, _EMBED), jnp.float32).at[:65].set(emb2)
  e1p = jnp.stack([emb1p[:, k:k + 121] for k in range(8)],
                  axis=1).reshape(64, 121)
  e2p = jnp.stack([emb2p[:, k:k + 121] for k in range(8)],
                  axis=1).reshape(576, 121)
  fw_oh = fc_xr_w.reshape(_OUT, _NF, 121).transpose(2, 1, 0).reshape(121,
                                                                     _NF * _OUT)
  xc_rna = _rna_mix(c1.reshape(_B, 8 * 256), c2.reshape(_B, 72 * 256),
                    e1p, e2p, fw_oh, fc_xr_b,
                    jnp.stack([conv1_b, conv2_b], axis=0))

  row = pro_edge_index[0]
  col = pro_edge_index[1]
  deg1 = jax.ops.segment_sum(pro_edge_weight, col, num_segments=_N) + 1.0
  deg23 = jax.ops.segment_sum(jnp.ones((_E,), jnp.float32), col,
                              num_segments=_N) + 1.0
  dinv1 = jax.lax.rsqrt(deg1).reshape(_N, 1)
  dinv23 = jax.lax.rsqrt(deg23).reshape(_N, 1)

  xs1 = _stage_xs(pro_x, gcn_w1.T, dinv1)
  agg1 = jax.ops.segment_sum(xs1[row] * pro_edge_weight[:, None], col,
                             num_segments=_N)
  x1 = _stage_post(agg1, xs1, dinv1, gcn_b1, bn1_w, bn1_b)

  xs2 = _stage_xs(x1, gcn_w2.T, dinv23)
  agg2 = jax.ops.segment_sum(xs2[row], col, num_segments=_N)
  x2 = _stage_post(agg2, xs2, dinv23, gcn_b2, bn2_w, bn2_b)

  xs3 = _stage_xs(x2, gcn_w3.T, dinv23)
  agg3 = jax.ops.segment_sum(xs3[row], col, num_segments=_N)
  x3 = _stage_post(agg3, xs3, dinv23, gcn_b3, bn3_w, bn3_b)

  sums_cat = _pool(pro_batch, x3)
  xp = _head(sums_cat, fc_g1_w.T, fc_g1_b, fc_g2_w.T, fc_g2_b)
  return (xc_rna, xp)


# speedup vs baseline: 8.1321x; 2.5580x over previous
import functools
import jax
import jax.numpy as jnp
from jax import lax
from jax.experimental import pallas as pl
from jax.experimental.pallas import tpu as pltpu
from jax.experimental.pallas import tpu_sc as plsc

_N = 50000
_E = 800000
_B = 32
_MAX_LEN = 3000
_LOCAL_LEN = 2998
_EMBED = 128
_NF = 32
_OUT = 128
_BM = 2000


def _onehot_body(g1_ref, g2_ref, w1_ref, w2_ref, c1_ref, c2_ref):
  g1 = g1_ref[0]
  oh1 = (jax.lax.broadcasted_iota(jnp.int32, (8, _MAX_LEN), 0)
         == g1).astype(jnp.float32)
  c1_ref[0] = jnp.dot(oh1, w1_ref[...], preferred_element_type=jnp.float32)
  g2 = g2_ref[0]
  oh2 = (jax.lax.broadcasted_iota(jnp.int32, (72, _LOCAL_LEN), 0)
         == g2).astype(jnp.float32)
  c2_ref[0] = jnp.dot(oh2, w2_ref[...], preferred_element_type=jnp.float32)


def _rna_counts(g1, g2, w1p, w2p):
  return pl.pallas_call(
      _onehot_body,
      grid=(_B,),
      in_specs=[
          pl.BlockSpec((1, 1, _MAX_LEN), lambda i: (i, 0, 0)),
          pl.BlockSpec((1, 1, _LOCAL_LEN), lambda i: (i, 0, 0)),
          pl.BlockSpec((_MAX_LEN, 256), lambda i: (0, 0)),
          pl.BlockSpec((_LOCAL_LEN, 256), lambda i: (0, 0)),
      ],
      out_specs=[
          pl.BlockSpec((1, 8, 256), lambda i: (i, 0, 0)),
          pl.BlockSpec((1, 72, 256), lambda i: (i, 0, 0)),
      ],
      out_shape=[
          jax.ShapeDtypeStruct((_B, 8, 256), jnp.float32),
          jax.ShapeDtypeStruct((_B, 72, 256), jnp.float32),
      ],
  )(g1.reshape(_B, 1, _MAX_LEN), g2.reshape(_B, 1, _LOCAL_LEN), w1p, w2p)


def _mix_body(c1_ref, c2_ref, e1_ref, e2_ref, fw_ref, b_ref, cb_ref, o_ref):
  m1 = jnp.dot(e1_ref[...], fw_ref[...],
               preferred_element_type=jnp.float32)
  m1 = m1.reshape(64 * 32, 128)
  xrg = jnp.dot(c1_ref[...], m1, preferred_element_type=jnp.float32)
  m2 = jnp.dot(e2_ref[...], fw_ref[...],
               preferred_element_type=jnp.float32)
  m2 = m2.reshape(576 * 32, 128)
  xrl = jnp.dot(c2_ref[...], m2, preferred_element_type=jnp.float32)
  fs = jnp.sum(fw_ref[...], axis=0).reshape(_NF, _OUT)
  cbt = jnp.dot(cb_ref[...], fs, preferred_element_type=jnp.float32)
  o_ref[...] = ((xrg + xrl) * 0.5 + b_ref[...]
                + 0.5 * (cbt[0:1, :] + cbt[1:2, :]))


def _rna_mix(c1f, c2f, e1p, e2p, fw_oh, fc_xr_b, cb):
  return pl.pallas_call(
      _mix_body,
      out_shape=jax.ShapeDtypeStruct((_B, _OUT), jnp.float32),
  )(c1f, c2f, e1p, e2p, fw_oh, fc_xr_b.reshape(1, _OUT), cb)


def _xs_body(x_ref, w_ref, d0_ref, d1_ref, xs_ref):
  xw = jnp.dot(x_ref[...], w_ref[...], preferred_element_type=jnp.float32)
  dinv = lax.rsqrt(d0_ref[...] + d1_ref[...] + 1.0)
  xs_ref[...] = xw * dinv


def _stage_xs(x, wT, d0, d1):
  fin, fout = wT.shape
  return pl.pallas_call(
      _xs_body,
      grid=(_N // _BM,),
      in_specs=[
          pl.BlockSpec((_BM, fin), lambda i: (i, 0)),
          pl.BlockSpec((fin, fout), lambda i: (0, 0)),
          pl.BlockSpec((_BM, 1), lambda i: (i, 0)),
          pl.BlockSpec((_BM, 1), lambda i: (i, 0)),
      ],
      out_specs=pl.BlockSpec((_BM, fout), lambda i: (i, 0)),
      out_shape=jax.ShapeDtypeStruct((_N, fout), jnp.float32),
  )(x, wT, d0, d1)


def _post_body(a0_ref, a1_ref, xs_ref, d0_ref, d1_ref, b_ref, bnw_ref,
               bnb_ref, o_ref):
  dinv = lax.rsqrt(d0_ref[...] + d1_ref[...] + 1.0)
  y = (a0_ref[...] + a1_ref[...] + xs_ref[...]) * dinv + b_ref[...]
  y = y * (1.0 / jnp.sqrt(1.0 + 1e-05)) * bnw_ref[...] + bnb_ref[...]
  o_ref[...] = jnp.maximum(y, 0.0)


def _stage_post(a0, a1, xs, d0, d1, b, bnw, bnb):
  f = xs.shape[1]
  return pl.pallas_call(
      _post_body,
      grid=(_N // _BM,),
      in_specs=[
          pl.BlockSpec((_BM, f), lambda i: (i, 0)),
          pl.BlockSpec((_BM, f), lambda i: (i, 0)),
          pl.BlockSpec((_BM, f), lambda i: (i, 0)),
          pl.BlockSpec((_BM, 1), lambda i: (i, 0)),
          pl.BlockSpec((_BM, 1), lambda i: (i, 0)),
          pl.BlockSpec((1, f), lambda i: (0, 0)),
          pl.BlockSpec((1, f), lambda i: (0, 0)),
          pl.BlockSpec((1, f), lambda i: (0, 0)),
      ],
      out_specs=pl.BlockSpec((_BM, f), lambda i: (i, 0)),
      out_shape=jax.ShapeDtypeStruct((_N, f), jnp.float32),
  )(a0, a1, xs, d0, d1, b.reshape(1, f), bnw.reshape(1, f),
    bnb.reshape(1, f))


_NC = 2
_NS = 16
_NW = _NC * _NS
_EPT = 25088
_NCH = 196
_EPAD = _NW * _EPT
_NPAD = 50048
_RPS = _NPAD // _NS


def _sc_mesh():
  return plsc.VectorSubcoreMesh(core_axis_name="c", subcore_axis_name="s",
                                num_cores=_NC, num_subcores=_NS)


def _sc_scatter(table, rowi, coli, ew, fc):
  scaled = ew is not None

  def body(*refs):
    if scaled:
      (table_h, rowi_h, coli_h, ew_h, zeros_h, out_h,
       rowv, colv, eww, rows_v, acc, sem) = refs
    else:
      (table_h, rowi_h, coli_h, zeros_h, out_h,
       rowv, colv, rows_v, acc, sem) = refs
    c = lax.axis_index("c")
    s = lax.axis_index("s")
    wid = s * _NC + c
    pltpu.sync_copy(rowi_h.at[wid], rowv)
    pltpu.sync_copy(coli_h.at[wid], colv)
    if scaled:
      pltpu.sync_copy(ew_h.at[wid], eww)
    pltpu.sync_copy(zeros_h, acc.at[pl.ds(s * _RPS, _RPS)])
    plsc.subcore_barrier()

    def chunk(j, carry):
      pltpu.async_copy(table_h.at[rowv.at[j]], rows_v, sem).wait()
      if scaled:
        for g in range(8):
          ewv = eww[j, pl.ds(16 * g, 16)]
          for l in range(16):
            i = 16 * g + l
            sc = ewv[l]
            for t in range(fc // 16):
              rows_v[i, pl.ds(16 * t, 16)] = rows_v[i, pl.ds(16 * t, 16)] * sc
      pltpu.sync_copy(rows_v, acc.at[colv.at[j]], add=True)
      return carry

    lax.fori_loop(0, _NCH, chunk, 0)
    plsc.subcore_barrier()
    pltpu.sync_copy(acc.at[pl.ds(s * _RPS, _RPS)],
                    out_h.at[c, pl.ds(s * _RPS, _RPS)])

  scratch = [
      pltpu.VMEM((_NCH, 128), jnp.int32),
      pltpu.VMEM((_NCH, 128), jnp.int32),
  ]
  if scaled:
    scratch.append(pltpu.VMEM((_NCH, 128), jnp.float32))
  scratch += [
      pltpu.VMEM((128, fc), jnp.float32),
      pltpu.VMEM_SHARED((_NPAD, fc), jnp.float32),
      pltpu.SemaphoreType.DMA,
  ]
  kern = pl.kernel(
      body,
      out_type=jax.ShapeDtypeStruct((_NC, _NPAD, fc), jnp.float32),
      mesh=_sc_mesh(),
      scratch_types=scratch,
      compiler_params=pltpu.CompilerParams(use_tc_tiling_on_sc=False),
  )
  zeros_h = jnp.zeros((_RPS, fc), jnp.float32)
  if scaled:
    return kern(table, rowi, coli, ew, zeros_h)
  return kern(table, rowi, coli, zeros_h)


def _sc_deg(coli, ew):
  def body(coli_h, ew_h, zeros_h, out_h, colv, eww, rows_v, acc, sem):
    c = lax.axis_index("c")
    s = lax.axis_index("s")
    wid = s * _NC + c
    pltpu.sync_copy(coli_h.at[wid], colv)
    pltpu.sync_copy(ew_h.at[wid], eww)
    pltpu.sync_copy(zeros_h, acc.at[pl.ds(s * _RPS, _RPS)])
    plsc.subcore_barrier()
    io = lax.iota(jnp.int32, 16)

    def chunk(j, carry):
      for g in range(8):
        ewv = eww[j, pl.ds(16 * g, 16)]
        for l in range(16):
          sc = ewv[l]
          rows_v[16 * g + l, :] = jnp.where(io == 0, sc,
                                            jnp.where(io == 1, 1.0, 0.0))
      pltpu.sync_copy(rows_v, acc.at[colv.at[j]], add=True)
      return carry

    lax.fori_loop(0, _NCH, chunk, 0)
    plsc.subcore_barrier()
    pltpu.sync_copy(acc.at[pl.ds(s * _RPS, _RPS)],
                    out_h.at[c, pl.ds(s * _RPS, _RPS)])

  kern = pl.kernel(
      body,
      out_type=jax.ShapeDtypeStruct((_NC, _NPAD, 16), jnp.float32),
      mesh=_sc_mesh(),
      scratch_types=[
          pltpu.VMEM((_NCH, 128), jnp.int32),
          pltpu.VMEM((_NCH, 128), jnp.float32),
          pltpu.VMEM((128, 16), jnp.float32),
          pltpu.VMEM_SHARED((_NPAD, 16), jnp.float32),
          pltpu.SemaphoreType.DMA,
      ],
      compiler_params=pltpu.CompilerParams(use_tc_tiling_on_sc=False),
  )
  return kern(coli, ew, jnp.zeros((_RPS, 16), jnp.float32))


def _edge_agg(xs, rowi, coli, ew, f):
  fpad = ((f + 15) // 16) * 16
  xs_p = jnp.zeros((_N, fpad), jnp.float32).at[:, :f].set(xs)
  parts = []
  c0 = 0
  while c0 < fpad:
    fc = 16
    parts.append(_sc_scatter(xs_p[:, c0:c0 + fc], rowi, coli, ew, fc))
    c0 += fc
  return jnp.concatenate(parts, axis=2)[:, :, :f] if len(parts) > 1 else \
      parts[0][:, :, :f]


def _pool_body(batch_ref, x_ref, acc_ref):
  i = pl.program_id(0)

  @pl.when(i == 0)
  def _():
    acc_ref[...] = jnp.zeros_like(acc_ref)

  bid = batch_ref[0]
  oh = (jax.lax.broadcasted_iota(jnp.int32, (_B, _BM), 0)
        == bid).astype(jnp.float32)
  xcat = jnp.concatenate(
      [x_ref[...], jnp.ones((_BM, 8), jnp.float32)], axis=1)
  acc_ref[...] += jnp.dot(oh, xcat, preferred_element_type=jnp.float32)


def _pool(batch, x3):
  f = x3.shape[1]
  return pl.pallas_call(
      _pool_body,
      grid=(_N // _BM,),
      in_specs=[
          pl.BlockSpec((1, 1, _BM), lambda i: (i, 0, 0)),
          pl.BlockSpec((_BM, f), lambda i: (i, 0)),
      ],
      out_specs=pl.BlockSpec((_B, f + 8), lambda i: (0, 0)),
      out_shape=jax.ShapeDtypeStruct((_B, f + 8), jnp.float32),
  )(batch.reshape(_N // _BM, 1, _BM), x3)


def _head_body(s_ref, w1_ref, b1_ref, w2_ref, b2_ref, o_ref):
  f = 132
  sums = s_ref[:, :f]
  cnt = jnp.maximum(s_ref[:, f:f + 1], 1.0)
  mean = sums / cnt
  h = jnp.dot(mean, w1_ref[...], preferred_element_type=jnp.float32)
  h = jnp.maximum(h + b1_ref[...], 0.0)
  o_ref[...] = jnp.dot(h, w2_ref[...],
                       preferred_element_type=jnp.float32) + b2_ref[...]


def _head(sums_cat, w1T, b1, w2T, b2):
  return pl.pallas_call(
      _head_body,
      out_shape=jax.ShapeDtypeStruct((_B, _OUT), jnp.float32),
  )(sums_cat, w1T, b1.reshape(1, -1), w2T, b2.reshape(1, -1))


def kernel(pro_x, pro_edge_index, pro_edge_weight, pro_batch, global_rna,
           local_rna, emb1, emb2, conv1_w, conv1_b, conv2_w, conv2_b,
           fc_xr_w, fc_xr_b, gcn_w1, gcn_b1, gcn_w2, gcn_b2, gcn_w3, gcn_b3,
           bn1_w, bn1_b, bn2_w, bn2_b, bn3_w, bn3_b, fc_g1_w, fc_g1_b,
           fc_g2_w, fc_g2_b):
  w1p = conv1_w.transpose(1, 2, 0).reshape(_MAX_LEN, 8 * _NF)
  w2p = conv2_w.transpose(1, 2, 0).reshape(_LOCAL_LEN, 8 * _NF)
  c1, c2 = _rna_counts(global_rna, local_rna, w1p, w2p)
  emb1p = jnp.zeros((8, _EMBED), jnp.float32).at[:5].set(emb1)
  emb2p = jnp.zeros((72, _EMBED), jnp.float32).at[:65].set(emb2)
  e1p = jnp.stack([emb1p[:, k:k + 121] for k in range(8)],
                  axis=1).reshape(64, 121)
  e2p = jnp.stack([emb2p[:, k:k + 121] for k in range(8)],
                  axis=1).reshape(576, 121)
  fw_oh = fc_xr_w.reshape(_OUT, _NF, 121).transpose(2, 1, 0).reshape(121,
                                                                     _NF * _OUT)
  xc_rna = _rna_mix(c1.reshape(_B, 8 * 256), c2.reshape(_B, 72 * 256),
                    e1p, e2p, fw_oh, fc_xr_b,
                    jnp.stack([conv1_b, conv2_b], axis=0))

  row = pro_edge_index[0]
  col = pro_edge_index[1]
  npad = _EPAD - _E
  rowi = jnp.concatenate([row, jnp.zeros((npad,), jnp.int32)]
                         ).reshape(_NW, _NCH, 128)
  coli = jnp.concatenate([col, jnp.full((npad,), _N, jnp.int32)]
                         ).reshape(_NW, _NCH, 128)
  ewi = jnp.concatenate([pro_edge_weight, jnp.zeros((npad,), jnp.float32)]
                        ).reshape(_NW, _NCH, 128)

  dparts = _sc_deg(coli, ewi)
  d1w = [dparts[0, :_N, 0:1], dparts[1, :_N, 0:1]]
  d1u = [dparts[0, :_N, 1:2], dparts[1, :_N, 1:2]]

  xs1 = _stage_xs(pro_x, gcn_w1.T, *d1w)
  p1 = _edge_agg(xs1, rowi, coli, ewi, 33)
  x1 = _stage_post(p1[0, :_N], p1[1, :_N], xs1, *d1w, gcn_b1, bn1_w, bn1_b)

  xs2 = _stage_xs(x1, gcn_w2.T, *d1u)
  p2 = _edge_agg(xs2, rowi, coli, None, 66)
  x2 = _stage_post(p2[0, :_N], p2[1, :_N], xs2, *d1u, gcn_b2, bn2_w, bn2_b)

  xs3 = _stage_xs(x2, gcn_w3.T, *d1u)
  p3 = _edge_agg(xs3, rowi, coli, None, 132)
  x3 = _stage_post(p3[0, :_N], p3[1, :_N], xs3, *d1u, gcn_b3, bn3_w, bn3_b)

  sums_cat = _pool(pro_batch, x3)
  xp = _head(sums_cat, fc_g1_w.T, fc_g1_b, fc_g2_w.T, fc_g2_b)
  return (xc_rna, xp)


# speedup vs baseline: 14.5625x; 1.7907x over previous
import functools
import jax
import jax.numpy as jnp
from jax import lax
from jax.experimental import pallas as pl
from jax.experimental.pallas import tpu as pltpu
from jax.experimental.pallas import tpu_sc as plsc

_N = 50000
_E = 800000
_B = 32
_MAX_LEN = 3000
_LOCAL_LEN = 2998
_EMBED = 128
_NF = 32
_OUT = 128
_BM = 2000


def _onehot_body(g1_ref, g2_ref, w1_ref, w2_ref, c1_ref, c2_ref):
  g1 = g1_ref[0]
  oh1 = (jax.lax.broadcasted_iota(jnp.int32, (8, _MAX_LEN), 0)
         == g1).astype(jnp.float32)
  c1_ref[0] = jnp.dot(oh1, w1_ref[...], preferred_element_type=jnp.float32)
  g2 = g2_ref[0]
  oh2 = (jax.lax.broadcasted_iota(jnp.int32, (72, _LOCAL_LEN), 0)
         == g2).astype(jnp.float32)
  c2_ref[0] = jnp.dot(oh2, w2_ref[...], preferred_element_type=jnp.float32)


def _rna_counts(g1, g2, w1p, w2p):
  return pl.pallas_call(
      _onehot_body,
      grid=(_B,),
      in_specs=[
          pl.BlockSpec((1, 1, _MAX_LEN), lambda i: (i, 0, 0)),
          pl.BlockSpec((1, 1, _LOCAL_LEN), lambda i: (i, 0, 0)),
          pl.BlockSpec((_MAX_LEN, 256), lambda i: (0, 0)),
          pl.BlockSpec((_LOCAL_LEN, 256), lambda i: (0, 0)),
      ],
      out_specs=[
          pl.BlockSpec((1, 8, 256), lambda i: (i, 0, 0)),
          pl.BlockSpec((1, 72, 256), lambda i: (i, 0, 0)),
      ],
      out_shape=[
          jax.ShapeDtypeStruct((_B, 8, 256), jnp.float32),
          jax.ShapeDtypeStruct((_B, 72, 256), jnp.float32),
      ],
  )(g1.reshape(_B, 1, _MAX_LEN), g2.reshape(_B, 1, _LOCAL_LEN), w1p, w2p)


def _mix_body(c1_ref, c2_ref, e1_ref, e2_ref, fw_ref, b_ref, cb_ref, o_ref):
  m1 = jnp.dot(e1_ref[...], fw_ref[...],
               preferred_element_type=jnp.float32)
  m1 = m1.reshape(64 * 32, 128)
  xrg = jnp.dot(c1_ref[...], m1, preferred_element_type=jnp.float32)
  m2 = jnp.dot(e2_ref[...], fw_ref[...],
               preferred_element_type=jnp.float32)
  m2 = m2.reshape(576 * 32, 128)
  xrl = jnp.dot(c2_ref[...], m2, preferred_element_type=jnp.float32)
  fs = jnp.sum(fw_ref[...], axis=0).reshape(_NF, _OUT)
  cbt = jnp.dot(cb_ref[...], fs, preferred_element_type=jnp.float32)
  o_ref[...] = ((xrg + xrl) * 0.5 + b_ref[...]
                + 0.5 * (cbt[0:1, :] + cbt[1:2, :]))


def _rna_mix(c1f, c2f, e1p, e2p, fw_oh, fc_xr_b, cb):
  return pl.pallas_call(
      _mix_body,
      out_shape=jax.ShapeDtypeStruct((_B, _OUT), jnp.float32),
  )(c1f, c2f, e1p, e2p, fw_oh, fc_xr_b.reshape(1, _OUT), cb)


def _xs_body(x_ref, w_ref, d0_ref, d1_ref, xs_ref):
  xw = jnp.dot(x_ref[...], w_ref[...], preferred_element_type=jnp.float32)
  dinv = lax.rsqrt(d0_ref[...] + d1_ref[...] + 1.0)
  xs_ref[...] = xw * dinv


def _stage_xs(x, wT, d0, d1):
  fin, fout = wT.shape
  return pl.pallas_call(
      _xs_body,
      grid=(_N // _BM,),
      in_specs=[
          pl.BlockSpec((_BM, fin), lambda i: (i, 0)),
          pl.BlockSpec((fin, fout), lambda i: (0, 0)),
          pl.BlockSpec((_BM, 1), lambda i: (i, 0)),
          pl.BlockSpec((_BM, 1), lambda i: (i, 0)),
      ],
      out_specs=pl.BlockSpec((_BM, fout), lambda i: (i, 0)),
      out_shape=jax.ShapeDtypeStruct((_N, fout), jnp.float32),
  )(x, wT, d0, d1)


def _post_body(a0_ref, a1_ref, xs_ref, d0_ref, d1_ref, b_ref, bnw_ref,
               bnb_ref, o_ref):
  dinv = lax.rsqrt(d0_ref[...] + d1_ref[...] + 1.0)
  y = (a0_ref[...] + a1_ref[...] + xs_ref[...]) * dinv + b_ref[...]
  y = y * (1.0 / jnp.sqrt(1.0 + 1e-05)) * bnw_ref[...] + bnb_ref[...]
  o_ref[...] = jnp.maximum(y, 0.0)


def _stage_post(a0, a1, xs, d0, d1, b, bnw, bnb):
  f = xs.shape[1]
  return pl.pallas_call(
      _post_body,
      grid=(_N // _BM,),
      in_specs=[
          pl.BlockSpec((_BM, f), lambda i: (i, 0)),
          pl.BlockSpec((_BM, f), lambda i: (i, 0)),
          pl.BlockSpec((_BM, f), lambda i: (i, 0)),
          pl.BlockSpec((_BM, 1), lambda i: (i, 0)),
          pl.BlockSpec((_BM, 1), lambda i: (i, 0)),
          pl.BlockSpec((1, f), lambda i: (0, 0)),
          pl.BlockSpec((1, f), lambda i: (0, 0)),
          pl.BlockSpec((1, f), lambda i: (0, 0)),
      ],
      out_specs=pl.BlockSpec((_BM, f), lambda i: (i, 0)),
      out_shape=jax.ShapeDtypeStruct((_N, f), jnp.float32),
  )(a0, a1, xs, d0, d1, b.reshape(1, f), bnw.reshape(1, f),
    bnb.reshape(1, f))


_NC = 2
_NS = 16
_NW = _NC * _NS
_EPT = 25088
_NCH = 196
_EPAD = _NW * _EPT
_NPAD = 50048
_RPS = _NPAD // _NS
_GB = 4
_QS = 28


def _sc_mesh():
  return plsc.VectorSubcoreMesh(core_axis_name="c", subcore_axis_name="s",
                                num_cores=_NC, num_subcores=_NS)


def _sc_scatter(table, rowi, coli, ew, fc):
  scaled = ew is not None

  def body(*refs):
    if scaled:
      (table_h, rowi_h, coli_h, ew_h, zeros_h, out_h,
       rowv, colv, eww, rows_v, acc, sem) = refs
    else:
      (table_h, rowi_h, coli_h, zeros_h, out_h,
       rowv, colv, rows_v, acc, sem) = refs
    c = lax.axis_index("c")
    s = lax.axis_index("s")
    wid = s * _NC + c
    pltpu.sync_copy(zeros_h, acc.at[pl.ds(s * _RPS, _RPS)])
    plsc.subcore_barrier()

    def qloop(q, carry):
      pltpu.sync_copy(rowi_h.at[wid, pl.ds(q * _QS, _QS)], rowv)
      pltpu.sync_copy(coli_h.at[wid, pl.ds(q * _QS, _QS)], colv)
      if scaled:
        pltpu.sync_copy(ew_h.at[wid, pl.ds(q * _QS, _QS)], eww)

      def chunk(jo, carry2):
        handles = []
        for b in range(_GB):
          handles.append(pltpu.async_copy(
              table_h.at[rowv.at[jo * _GB + b]],
              rows_v.at[pl.ds(b * 128, 128)], sem))
        for h in handles:
          h.wait()
        for b in range(_GB):
          if scaled:
            for g in range(8):
              ewv = eww[jo * _GB + b, pl.ds(16 * g, 16)]
              for l in range(16):
                i = b * 128 + 16 * g + l
                sc = ewv[l]
                for t in range(fc // 16):
                  rows_v[i, pl.ds(16 * t, 16)] = (
                      rows_v[i, pl.ds(16 * t, 16)] * sc)
          pltpu.sync_copy(rows_v.at[pl.ds(b * 128, 128)],
                          acc.at[colv.at[jo * _GB + b]], add=True)
        return carry2

      lax.fori_loop(0, _QS // _GB, chunk, 0)
      return carry

    lax.fori_loop(0, _NCH // _QS, qloop, 0)
    plsc.subcore_barrier()
    pltpu.sync_copy(acc.at[pl.ds(s * _RPS, _RPS)],
                    out_h.at[c, pl.ds(s * _RPS, _RPS)])

  scratch = [
      pltpu.VMEM((_QS, 128), jnp.int32),
      pltpu.VMEM((_QS, 128), jnp.int32),
  ]
  if scaled:
    scratch.append(pltpu.VMEM((_QS, 128), jnp.float32))
  scratch += [
      pltpu.VMEM((_GB * 128, fc), jnp.float32),
      pltpu.VMEM_SHARED((_NPAD, fc), jnp.float32),
      pltpu.SemaphoreType.DMA,
  ]
  kern = pl.kernel(
      body,
      out_type=jax.ShapeDtypeStruct((_NC, _NPAD, fc), jnp.float32),
      mesh=_sc_mesh(),
      scratch_types=scratch,
      compiler_params=pltpu.CompilerParams(use_tc_tiling_on_sc=False),
  )
  zeros_h = jnp.zeros((_RPS, fc), jnp.float32)
  if scaled:
    return kern(table, rowi, coli, ew, zeros_h)
  return kern(table, rowi, coli, zeros_h)


def _sc_deg(coli, ew):
  def body(coli_h, ew_h, zeros_h, out_h, colv, eww, rows_v, acc, sem):
    c = lax.axis_index("c")
    s = lax.axis_index("s")
    wid = s * _NC + c
    pltpu.sync_copy(coli_h.at[wid], colv)
    pltpu.sync_copy(ew_h.at[wid], eww)
    pltpu.sync_copy(zeros_h, acc.at[pl.ds(s * _RPS, _RPS)])
    plsc.subcore_barrier()
    io = lax.iota(jnp.int32, 16)

    def chunk(j, carry):
      for g in range(8):
        ewv = eww[j, pl.ds(16 * g, 16)]
        for l in range(16):
          sc = ewv[l]
          rows_v[16 * g + l, :] = jnp.where(io == 0, sc,
                                            jnp.where(io == 1, 1.0, 0.0))
      pltpu.sync_copy(rows_v, acc.at[colv.at[j]], add=True)
      return carry

    lax.fori_loop(0, _NCH, chunk, 0)
    plsc.subcore_barrier()
    pltpu.sync_copy(acc.at[pl.ds(s * _RPS, _RPS)],
                    out_h.at[c, pl.ds(s * _RPS, _RPS)])

  kern = pl.kernel(
      body,
      out_type=jax.ShapeDtypeStruct((_NC, _NPAD, 16), jnp.float32),
      mesh=_sc_mesh(),
      scratch_types=[
          pltpu.VMEM((_NCH, 128), jnp.int32),
          pltpu.VMEM((_NCH, 128), jnp.float32),
          pltpu.VMEM((128, 16), jnp.float32),
          pltpu.VMEM_SHARED((_NPAD, 16), jnp.float32),
          pltpu.SemaphoreType.DMA,
      ],
      compiler_params=pltpu.CompilerParams(use_tc_tiling_on_sc=False),
  )
  return kern(coli, ew, jnp.zeros((_RPS, 16), jnp.float32))


def _edge_agg(xs, rowi, coli, ew, f):
  fpad = ((f + 15) // 16) * 16
  xs_p = jnp.zeros((_N, fpad), jnp.float32).at[:, :f].set(xs)
  parts = []
  c0 = 0
  while c0 < fpad:
    fc = 32 if fpad - c0 >= 32 else 16
    parts.append(_sc_scatter(xs_p[:, c0:c0 + fc], rowi, coli, ew, fc))
    c0 += fc
  return jnp.concatenate(parts, axis=2)[:, :, :f] if len(parts) > 1 else \
      parts[0][:, :, :f]


def _pool_body(batch_ref, x_ref, acc_ref):
  i = pl.program_id(0)

  @pl.when(i == 0)
  def _():
    acc_ref[...] = jnp.zeros_like(acc_ref)

  bid = batch_ref[0]
  oh = (jax.lax.broadcasted_iota(jnp.int32, (_B, _BM), 0)
        == bid).astype(jnp.float32)
  xcat = jnp.concatenate(
      [x_ref[...], jnp.ones((_BM, 8), jnp.float32)], axis=1)
  acc_ref[...] += jnp.dot(oh, xcat, preferred_element_type=jnp.float32)


def _pool(batch, x3):
  f = x3.shape[1]
  return pl.pallas_call(
      _pool_body,
      grid=(_N // _BM,),
      in_specs=[
          pl.BlockSpec((1, 1, _BM), lambda i: (i, 0, 0)),
          pl.BlockSpec((_BM, f), lambda i: (i, 0)),
      ],
      out_specs=pl.BlockSpec((_B, f + 8), lambda i: (0, 0)),
      out_shape=jax.ShapeDtypeStruct((_B, f + 8), jnp.float32),
  )(batch.reshape(_N // _BM, 1, _BM), x3)


def _head_body(s_ref, w1_ref, b1_ref, w2_ref, b2_ref, o_ref):
  f = 132
  sums = s_ref[:, :f]
  cnt = jnp.maximum(s_ref[:, f:f + 1], 1.0)
  mean = sums / cnt
  h = jnp.dot(mean, w1_ref[...], preferred_element_type=jnp.float32)
  h = jnp.maximum(h + b1_ref[...], 0.0)
  o_ref[...] = jnp.dot(h, w2_ref[...],
                       preferred_element_type=jnp.float32) + b2_ref[...]


def _head(sums_cat, w1T, b1, w2T, b2):
  return pl.pallas_call(
      _head_body,
      out_shape=jax.ShapeDtypeStruct((_B, _OUT), jnp.float32),
  )(sums_cat, w1T, b1.reshape(1, -1), w2T, b2.reshape(1, -1))


def kernel(pro_x, pro_edge_index, pro_edge_weight, pro_batch, global_rna,
           local_rna, emb1, emb2, conv1_w, conv1_b, conv2_w, conv2_b,
           fc_xr_w, fc_xr_b, gcn_w1, gcn_b1, gcn_w2, gcn_b2, gcn_w3, gcn_b3,
           bn1_w, bn1_b, bn2_w, bn2_b, bn3_w, bn3_b, fc_g1_w, fc_g1_b,
           fc_g2_w, fc_g2_b):
  w1p = conv1_w.transpose(1, 2, 0).reshape(_MAX_LEN, 8 * _NF)
  w2p = conv2_w.transpose(1, 2, 0).reshape(_LOCAL_LEN, 8 * _NF)
  c1, c2 = _rna_counts(global_rna, local_rna, w1p, w2p)
  emb1p = jnp.zeros((8, _EMBED), jnp.float32).at[:5].set(emb1)
  emb2p = jnp.zeros((72, _EMBED), jnp.float32).at[:65].set(emb2)
  e1p = jnp.stack([emb1p[:, k:k + 121] for k in range(8)],
                  axis=1).reshape(64, 121)
  e2p = jnp.stack([emb2p[:, k:k + 121] for k in range(8)],
                  axis=1).reshape(576, 121)
  fw_oh = fc_xr_w.reshape(_OUT, _NF, 121).transpose(2, 1, 0).reshape(121,
                                                                     _NF * _OUT)
  xc_rna = _rna_mix(c1.reshape(_B, 8 * 256), c2.reshape(_B, 72 * 256),
                    e1p, e2p, fw_oh, fc_xr_b,
                    jnp.stack([conv1_b, conv2_b], axis=0))

  row = pro_edge_index[0]
  col = pro_edge_index[1]
  npad = _EPAD - _E
  rowi = jnp.concatenate([row, jnp.zeros((npad,), jnp.int32)]
                         ).reshape(_NW, _NCH, 128)
  coli = jnp.concatenate([col, jnp.full((npad,), _N, jnp.int32)]
                         ).reshape(_NW, _NCH, 128)
  ewi = jnp.concatenate([pro_edge_weight, jnp.zeros((npad,), jnp.float32)]
                        ).reshape(_NW, _NCH, 128)

  dparts = _sc_deg(coli, ewi)
  d1w = [dparts[0, :_N, 0:1], dparts[1, :_N, 0:1]]
  d1u = [dparts[0, :_N, 1:2], dparts[1, :_N, 1:2]]

  xs1 = _stage_xs(pro_x, gcn_w1.T, *d1w)
  p1 = _edge_agg(xs1, rowi, coli, ewi, 33)
  x1 = _stage_post(p1[0, :_N], p1[1, :_N], xs1, *d1w, gcn_b1, bn1_w, bn1_b)

  xs2 = _stage_xs(x1, gcn_w2.T, *d1u)
  p2 = _edge_agg(xs2, rowi, coli, None, 66)
  x2 = _stage_post(p2[0, :_N], p2[1, :_N], xs2, *d1u, gcn_b2, bn2_w, bn2_b)

  xs3 = _stage_xs(x2, gcn_w3.T, *d1u)
  p3 = _edge_agg(xs3, rowi, coli, None, 132)
  x3 = _stage_post(p3[0, :_N], p3[1, :_N], xs3, *d1u, gcn_b3, bn3_w, bn3_b)

  sums_cat = _pool(pro_batch, x3)
  xp = _head(sums_cat, fc_g1_w.T, fc_g1_b, fc_g2_w.T, fc_g2_b)
  return (xc_rna, xp)
